# baseline (speedup 1.0000x reference)
import jax
import jax.numpy as jnp
from jax import lax
from jax.experimental import pallas as pl
from jax.experimental.pallas import tpu as pltpu
from jax.experimental.pallas import tpu_sc as plsc

_W = 128
_NBUF = 5
_GD = 4
_NC, _NS = 2, 16


def kernel(x, vocab):
    B, S = x.shape
    V, D = vocab.shape
    N = B * S
    NW = _NC * _NS
    CH = N // (NW * _W)
    assert N == CH * NW * _W and CH % _NBUF == 0 and CH >= 2 * _NBUF

    idx = jnp.transpose(x).reshape(NW, CH, _W).astype(jnp.int32)
    mesh = plsc.VectorSubcoreMesh(core_axis_name="core", subcore_axis_name="subcore")

    @pl.kernel(
        out_type=jax.ShapeDtypeStruct((N, D), vocab.dtype),
        mesh=mesh,
        scratch_types=[
            pltpu.VMEM((CH, _W), jnp.int32),
            pltpu.VMEM((_NBUF, _W, D), jnp.float32),
            pltpu.SemaphoreType.DMA,
            pltpu.SemaphoreType.DMA((_NBUF,)),
            pltpu.SemaphoreType.DMA((_NBUF,)),
        ],
    )
    def gather_kernel(vocab_hbm, idx_hbm, out_hbm, idxbuf, rows, isem, gsem, osem):
        c = lax.axis_index("core")
        s = lax.axis_index("subcore")
        wid = s * _NC + c
        cbase = wid * CH

        pltpu.async_copy(idx_hbm.at[wid], idxbuf, isem).wait()

        def start_gather(j, b):
            pltpu.async_copy(vocab_hbm.at[idxbuf.at[j]], rows.at[b], gsem.at[b])

        def wait_gather(j, b):
            pltpu.make_async_copy(
                vocab_hbm.at[idxbuf.at[j]], rows.at[b], gsem.at[b]
            ).wait()

        def out_slice(j):
            off = pl.multiple_of((cbase + j) * _W, _W)
            return out_hbm.at[pl.ds(off, _W)]

        def start_out(j, b):
            pltpu.async_copy(rows.at[b], out_slice(j), osem.at[b])

        def wait_out(j, b):
            pltpu.make_async_copy(rows.at[b], out_slice(j), osem.at[b]).wait()

        for j in range(_GD):
            start_gather(j, j)
        for j in range(_NBUF):
            b = j
            bn = (b + _GD) % _NBUF
            wait_gather(j, b)
            if j >= _NBUF - _GD:
                wait_out(j - (_NBUF - _GD), bn)
            start_gather(j + _GD, bn)
            start_out(j, b)

        @pl.loop(_NBUF, CH - _NBUF, step=_NBUF)
        def _(j0):
            for b in range(_NBUF):
                j = j0 + b
                bn = (b + _GD) % _NBUF
                wait_gather(j, b)
                wait_out(j - (_NBUF - _GD), bn)
                start_gather(j + _GD, bn)
                start_out(j, b)

        for j in range(CH - _NBUF, CH):
            b = j % _NBUF
            bn = (b + _GD) % _NBUF
            wait_gather(j, b)
            if j + _GD < CH:
                wait_out(j - (_NBUF - _GD), bn)
                start_gather(j + _GD, bn)
            start_out(j, b)
        for j in range(CH - _NBUF, CH):
            wait_out(j, j % _NBUF)

    out_sb = gather_kernel(vocab, idx).reshape(S, B, D)
    return jnp.transpose(out_sb, (1, 0, 2))


# speedup vs baseline: 1.0189x; 1.0189x over previous
import jax
import jax.numpy as jnp
from jax import lax
from jax.experimental import pallas as pl
from jax.experimental.pallas import tpu as pltpu
from jax.experimental.pallas import tpu_sc as plsc

_W = 128
_NBUF = 5
_GD = 4
_NC, _NS = 2, 16


def kernel(x, vocab):
    B, S = x.shape
    V, D = vocab.shape
    N = B * S
    NW = _NC * _NS
    CH = S
    assert B == NW * _W and CH % _NBUF == 0 and CH >= 2 * _NBUF

    xt = jnp.transpose(x).astype(jnp.int32)
    mesh = plsc.VectorSubcoreMesh(core_axis_name="core", subcore_axis_name="subcore")

    @pl.kernel(
        out_type=jax.ShapeDtypeStruct((N, D), vocab.dtype),
        mesh=mesh,
        compiler_params=pltpu.CompilerParams(use_tc_tiling_on_sc=True),
        scratch_types=[
            pltpu.VMEM((CH, _W), jnp.int32),
            pltpu.VMEM((_NBUF, _W, D), jnp.float32),
            pltpu.SemaphoreType.DMA,
            pltpu.SemaphoreType.DMA((_NBUF,)),
            pltpu.SemaphoreType.DMA((_NBUF,)),
        ],
    )
    def gather_kernel(vocab_hbm, idx_hbm, out_hbm, idxbuf, rows, isem, gsem, osem):
        c = lax.axis_index("core")
        s = lax.axis_index("subcore")
        wid = s * _NC + c
        col = pl.multiple_of(wid * _W, _W)

        pltpu.async_copy(idx_hbm.at[:, pl.ds(col, _W)], idxbuf, isem).wait()

        def start_gather(j, b):
            pltpu.async_copy(vocab_hbm.at[idxbuf.at[j]], rows.at[b], gsem.at[b])

        def wait_gather(j, b):
            pltpu.make_async_copy(
                vocab_hbm.at[idxbuf.at[j]], rows.at[b], gsem.at[b]
            ).wait()

        def out_slice(j):
            off = pl.multiple_of(j * B + col, _W)
            return out_hbm.at[pl.ds(off, _W)]

        def start_out(j, b):
            pltpu.async_copy(rows.at[b], out_slice(j), osem.at[b])

        def wait_out(j, b):
            pltpu.make_async_copy(rows.at[b], out_slice(j), osem.at[b]).wait()

        for j in range(_GD):
            start_gather(j, j)
        for j in range(_NBUF):
            b = j
            bn = (b + _GD) % _NBUF
            wait_gather(j, b)
            if j >= _NBUF - _GD:
                wait_out(j - (_NBUF - _GD), bn)
            start_gather(j + _GD, bn)
            start_out(j, b)

        @pl.loop(_NBUF, CH - _NBUF, step=_NBUF)
        def _(j0):
            for b in range(_NBUF):
                j = j0 + b
                bn = (b + _GD) % _NBUF
                wait_gather(j, b)
                wait_out(j - (_NBUF - _GD), bn)
                start_gather(j + _GD, bn)
                start_out(j, b)

        for j in range(CH - _NBUF, CH):
            b = j % _NBUF
            bn = (b + _GD) % _NBUF
            wait_gather(j, b)
            if j + _GD < CH:
                wait_out(j - (_NBUF - _GD), bn)
                start_gather(j + _GD, bn)
            start_out(j, b)
        for j in range(CH - _NBUF, CH):
            wait_out(j, j % _NBUF)

    out_sb = gather_kernel(vocab, xt).reshape(S, B, D)
    return jnp.transpose(out_sb, (1, 0, 2))


# speedup vs baseline: 1.0219x; 1.0029x over previous
import jax
import jax.numpy as jnp
from jax import lax
from jax.experimental import pallas as pl
from jax.experimental.pallas import tpu as pltpu
from jax.experimental.pallas import tpu_sc as plsc

_W = 128
_NBUF = 5
_GD = 4
_NC, _NS = 2, 16


def kernel(x, vocab):
    B, S = x.shape
    V, D = vocab.shape
    N = B * S
    NW = _NC * _NS
    CH = S
    assert B == NW * _W and CH % _NBUF == 0 and CH >= 2 * _NBUF

    xt = jnp.transpose(x).astype(jnp.int32)
    mesh = plsc.VectorSubcoreMesh(core_axis_name="core", subcore_axis_name="subcore")

    @pl.kernel(
        out_type=jax.ShapeDtypeStruct((N, D), vocab.dtype),
        mesh=mesh,
        compiler_params=pltpu.CompilerParams(use_tc_tiling_on_sc=True),
        scratch_types=[
            pltpu.VMEM((CH, _W), jnp.int32),
            pltpu.VMEM((_NBUF, _W, D), jnp.float32),
            pltpu.SemaphoreType.DMA,
            pltpu.SemaphoreType.DMA((_NBUF,)),
            pltpu.SemaphoreType.DMA((_NBUF,)),
        ],
    )
    def gather_kernel(vocab_hbm, idx_hbm, out_hbm, idxbuf, rows, isem, gsem, osem):
        c = lax.axis_index("core")
        s = lax.axis_index("subcore")
        wid = s * _NC + c
        col = pl.multiple_of(wid * _W, _W)

        pltpu.async_copy(idx_hbm.at[:, pl.ds(col, _W)], idxbuf, isem).wait()

        def start_gather(j, b):
            pltpu.async_copy(vocab_hbm.at[idxbuf.at[j]], rows.at[b], gsem.at[b])

        def wait_gather(j, b):
            pltpu.make_async_copy(
                vocab_hbm.at[idxbuf.at[j]], rows.at[b], gsem.at[b]
            ).wait()

        def out_slice(j):
            off = pl.multiple_of(j * B + col, _W)
            return out_hbm.at[pl.ds(off, _W)]

        def start_out(j, b):
            pltpu.async_copy(rows.at[b], out_slice(j), osem.at[b])

        def wait_out(j, b):
            pltpu.make_async_copy(rows.at[b], out_slice(j), osem.at[b]).wait()

        for j in range(_GD):
            start_gather(j, j)
        for j in range(_NBUF):
            b = j
            bn = (b + _GD) % _NBUF
            if j >= _NBUF - _GD:
                wait_out(j - (_NBUF - _GD), bn)
            start_gather(j + _GD, bn)
            wait_gather(j, b)
            start_out(j, b)

        @pl.loop(_NBUF, CH - _NBUF, step=_NBUF)
        def _(j0):
            for b in range(_NBUF):
                j = j0 + b
                bn = (b + _GD) % _NBUF
                wait_out(j - (_NBUF - _GD), bn)
                start_gather(j + _GD, bn)
                wait_gather(j, b)
                start_out(j, b)

        for j in range(CH - _NBUF, CH):
            b = j % _NBUF
            bn = (b + _GD) % _NBUF
            if j + _GD < CH:
                wait_out(j - (_NBUF - _GD), bn)
                start_gather(j + _GD, bn)
            wait_gather(j, b)
            start_out(j, b)
        for j in range(CH - _NBUF, CH):
            wait_out(j, j % _NBUF)

    out_sb = gather_kernel(vocab, xt).reshape(S, B, D)
    return jnp.transpose(out_sb, (1, 0, 2))
